# baseline (speedup 1.0000x reference)
import jax
import jax.numpy as jnp
from jax.experimental import pallas as pl
from jax.experimental.pallas import tpu as pltpu

_N = 10000
_BJ = 400


def _s1_kernel(feature_ref, w1_ref, s1_ref):
    f = feature_ref[...].astype(jnp.bfloat16)
    w = w1_ref[...].astype(jnp.bfloat16)
    s1_ref[...] = jnp.dot(f, w, preferred_element_type=jnp.float32)


def _layer1_kernel(adj_ref, s1_ref, b1_ref, w2_ref, s2_ref):
    a = adj_ref[...].astype(jnp.bfloat16)
    x = jnp.dot(a, s1_ref[...], preferred_element_type=jnp.float32)
    x = jnp.maximum(x + b1_ref[...], 0.0)
    s2_ref[...] = jnp.dot(
        x.astype(jnp.bfloat16), w2_ref[...], preferred_element_type=jnp.float32
    )


def _layer2_kernel(adj_ref, s2_ref, b2_ref, out_ref):
    a = adj_ref[...].astype(jnp.bfloat16)
    out_ref[...] = (
        jnp.dot(a, s2_ref[...], preferred_element_type=jnp.float32) + b2_ref[...]
    )


@jax.jit
def kernel(feature, adj, W1, b1, W2, b2):
    n, nfeat = feature.shape
    nh1 = W1.shape[1]
    nh2 = W2.shape[1]
    b1r = b1.reshape(1, nh1)
    b2r = b2.reshape(1, nh2)

    s1 = pl.pallas_call(
        _s1_kernel,
        out_shape=jax.ShapeDtypeStruct((n, nh1), jnp.float32),
    )(feature, W1)
    s1 = s1.astype(jnp.bfloat16)

    grid = (n // _BJ,)
    s2 = pl.pallas_call(
        _layer1_kernel,
        grid=grid,
        in_specs=[
            pl.BlockSpec((_BJ, n), lambda j: (j, 0)),
            pl.BlockSpec((n, nh1), lambda j: (0, 0)),
            pl.BlockSpec((1, nh1), lambda j: (0, 0)),
            pl.BlockSpec((nh1, nh2), lambda j: (0, 0)),
        ],
        out_specs=pl.BlockSpec((_BJ, nh2), lambda j: (j, 0)),
        out_shape=jax.ShapeDtypeStruct((n, nh2), jnp.float32),
        compiler_params=pltpu.CompilerParams(
            dimension_semantics=("arbitrary",),
        ),
    )(adj, s1, b1r, W2.astype(jnp.bfloat16))
    s2 = s2.astype(jnp.bfloat16)

    out = pl.pallas_call(
        _layer2_kernel,
        grid=grid,
        in_specs=[
            pl.BlockSpec((_BJ, n), lambda j: (j, 0)),
            pl.BlockSpec((n, nh2), lambda j: (0, 0)),
            pl.BlockSpec((1, nh2), lambda j: (0, 0)),
        ],
        out_specs=pl.BlockSpec((_BJ, nh2), lambda j: (j, 0)),
        out_shape=jax.ShapeDtypeStruct((n, nh2), jnp.float32),
        compiler_params=pltpu.CompilerParams(
            dimension_semantics=("arbitrary",),
        ),
    )(adj, s2, b2r)
    return out
